# baseline (speedup 1.0000x reference)
import functools

import jax
import jax.numpy as jnp
from jax import lax
from jax.experimental import pallas as pl
from jax.experimental.pallas import tpu as pltpu, tpu_sc as plsc

NUM_VERBS = 25
NUM_HOIS = 234
BATCH = 16384

NC = 2
NS = 16
NW = NC * NS
LANES = 16

ROWS_PER_W = BATCH // NW
CHUNK = 128
NCHUNK = ROWS_PER_W // CHUNK
NGROUPS = -(-NUM_HOIS // LANES)
IDX_PAD = NGROUPS * LANES
OUT_WORDS = CHUNK * NUM_HOIS


def _sc_gather(in_hbm, idx_hbm, out_hbm, in_v, idx_v, out_v0, out_v1, sem0, sem1):
    wid = lax.axis_index("s") * NC + lax.axis_index("c")
    in_base = wid * (ROWS_PER_W * NUM_VERBS)
    out_base = wid * (ROWS_PER_W * NUM_HOIS)

    pltpu.sync_copy(idx_hbm, idx_v)
    pltpu.sync_copy(in_hbm.at[pl.ds(in_base, ROWS_PER_W * NUM_VERBS)], in_v)

    idxs = [idx_v[pl.ds(g * LANES, LANES)] for g in range(NGROUPS)]

    sems = (sem0, sem1)
    bufs = (out_v0, out_v1)
    copies = [None, None]
    for c in range(NCHUNK):
        buf = c % 2
        if copies[buf] is not None:
            copies[buf].wait()
        out_buf = bufs[buf]
        chunk_base = c * CHUNK

        def row_body(r, _, out_buf=out_buf, chunk_base=chunk_base):
            src = (chunk_base + r) * NUM_VERBS
            dst = r * NUM_HOIS
            for g in range(NGROUPS):
                vals = plsc.load_gather(in_v, [idxs[g] + src])
                out_buf[pl.ds(dst + g * LANES, LANES)] = vals
            return 0

        lax.fori_loop(0, CHUNK, row_body, 0)

        cp = pltpu.make_async_copy(
            out_buf.at[pl.ds(0, OUT_WORDS)],
            out_hbm.at[pl.ds(out_base + c * OUT_WORDS, OUT_WORDS)],
            sems[buf],
        )
        cp.start()
        copies[buf] = cp
    copies[0].wait()
    copies[1].wait()


@jax.jit
def kernel(verb_scores, hoi_to_verb):
    idx_pad = jnp.zeros((IDX_PAD,), jnp.int32).at[:NUM_HOIS].set(hoi_to_verb)
    mesh = plsc.VectorSubcoreMesh(
        core_axis_name="c", subcore_axis_name="s", num_cores=NC, num_subcores=NS
    )
    run = pl.kernel(
        _sc_gather,
        out_type=jax.ShapeDtypeStruct((BATCH * NUM_HOIS,), jnp.float32),
        mesh=mesh,
        scratch_types=[
            pltpu.VMEM((ROWS_PER_W * NUM_VERBS,), jnp.float32),
            pltpu.VMEM((IDX_PAD,), jnp.int32),
            pltpu.VMEM((OUT_WORDS + LANES,), jnp.float32),
            pltpu.VMEM((OUT_WORDS + LANES,), jnp.float32),
            pltpu.SemaphoreType.DMA,
            pltpu.SemaphoreType.DMA,
        ],
        compiler_params=pltpu.CompilerParams(needs_layout_passes=False),
    )
    out_flat = run(verb_scores.reshape(-1), idx_pad)
    return out_flat.reshape(BATCH, NUM_HOIS)


# speedup vs baseline: 3.1926x; 3.1926x over previous
import jax
import jax.numpy as jnp
from jax import lax
from jax.experimental import pallas as pl
from jax.experimental.pallas import tpu as pltpu

NUM_VERBS = 25
NUM_HOIS = 234
BATCH = 16384
BLOCK_B = 2048


def _gather_via_onehot(idx_ref, in_ref, out_ref):
    verb_iota = lax.broadcasted_iota(jnp.int32, (NUM_VERBS, NUM_HOIS), 0)
    onehot = (idx_ref[0][None, :] == verb_iota).astype(jnp.float32)
    out_ref[...] = jnp.dot(
        in_ref[...], onehot, preferred_element_type=jnp.float32
    )


@jax.jit
def kernel(verb_scores, hoi_to_verb):
    grid = (BATCH // BLOCK_B,)
    return pl.pallas_call(
        _gather_via_onehot,
        grid=grid,
        in_specs=[
            pl.BlockSpec((1, NUM_HOIS), lambda i: (0, 0)),
            pl.BlockSpec((BLOCK_B, NUM_VERBS), lambda i: (i, 0)),
        ],
        out_specs=pl.BlockSpec((BLOCK_B, NUM_HOIS), lambda i: (i, 0)),
        out_shape=jax.ShapeDtypeStruct((BATCH, NUM_HOIS), jnp.float32),
        compiler_params=pltpu.CompilerParams(
            dimension_semantics=("parallel",),
        ),
    )(hoi_to_verb.reshape(1, NUM_HOIS), verb_scores)
